# baseline (speedup 1.0000x reference)
import jax
import jax.numpy as jnp
from jax import lax
from jax.experimental import pallas as pl
from jax.experimental.pallas import tpu as pltpu


def _round_up(v, m):
    return ((v + m - 1) // m) * m


def _pad2(a, rows, cols):
    pr, pc = rows - a.shape[0], cols - a.shape[1]
    if pr or pc:
        a = jnp.pad(a, ((0, pr), (0, pc)))
    return a


def _weight_body(b_ref, a_ref, c_ref, w_ref, ab_ref):
    n = pl.program_id(0)

    @pl.when(n == 0)
    def _prep():
        ab_ref[...] = a_ref[...].astype(jnp.bfloat16)

    low = lax.dot_general(
        b_ref[...].astype(jnp.bfloat16), ab_ref[...],
        dimension_numbers=(((1,), (0,)), ((), ())),
        preferred_element_type=jnp.float32,
    )
    w_ref[...] = (c_ref[...] + low).astype(jnp.bfloat16)


def _gemm_body(x_ref, w_ref, bias_ref, o_ref, xs_ref):
    j = pl.program_id(1)

    @pl.when(j == 0)
    def _cast_x():
        xs_ref[...] = x_ref[...].astype(jnp.bfloat16)

    o_ref[...] = lax.dot_general(
        xs_ref[...], w_ref[...],
        dimension_numbers=(((1,), (1,)), ((), ())),
        preferred_element_type=jnp.float32,
    ) + bias_ref[...]


def kernel(x, weightA, weightB, weightC, bias):
    batch, in_f = x.shape
    out_f, rank = weightB.shape
    out_dtype = x.dtype

    tm = min(1024, _round_up(batch, 8))
    tn = min(512, _round_up(out_f, 128))
    tw = min(1024, _round_up(out_f, 128))
    M = _round_up(batch, tm)
    N = _round_up(out_f, tn)
    K = _round_up(in_f, 128)
    R = _round_up(rank, 128)

    x_p = _pad2(x, M, K)
    a_p = _pad2(weightA, R, K)
    c_p = _pad2(weightC, N, K)
    b_p = _pad2(weightB, N, R)
    bias_p = _pad2(bias.reshape(1, out_f).astype(jnp.float32), 1, N)

    w_eff = pl.pallas_call(
        _weight_body,
        out_shape=jax.ShapeDtypeStruct((N, K), jnp.bfloat16),
        grid=(N // tw,),
        in_specs=[
            pl.BlockSpec((tw, R), lambda n: (n, 0)),
            pl.BlockSpec((R, K), lambda n: (0, 0)),
            pl.BlockSpec((tw, K), lambda n: (n, 0)),
        ],
        out_specs=pl.BlockSpec((tw, K), lambda n: (n, 0)),
        scratch_shapes=[
            pltpu.VMEM((R, K), jnp.bfloat16),
        ],
        compiler_params=pltpu.CompilerParams(
            dimension_semantics=("arbitrary",),
            vmem_limit_bytes=56 * 1024 * 1024,
        ),
    )(b_p, a_p, c_p)

    out = pl.pallas_call(
        _gemm_body,
        out_shape=jax.ShapeDtypeStruct((M, N), out_dtype),
        grid=(M // tm, N // tn),
        in_specs=[
            pl.BlockSpec((tm, K), lambda i, j: (i, 0)),
            pl.BlockSpec((tn, K), lambda i, j: (j, 0)),
            pl.BlockSpec((1, tn), lambda i, j: (0, j)),
        ],
        out_specs=pl.BlockSpec((tm, tn), lambda i, j: (i, j)),
        scratch_shapes=[
            pltpu.VMEM((tm, K), jnp.bfloat16),
        ],
        compiler_params=pltpu.CompilerParams(
            dimension_semantics=("parallel", "arbitrary"),
            vmem_limit_bytes=56 * 1024 * 1024,
        ),
    )(x_p, w_eff, bias_p)

    if M != batch or N != out_f:
        out = out[:batch, :out_f]
    return out


# speedup vs baseline: 1.0037x; 1.0037x over previous
import jax
import jax.numpy as jnp
from jax import lax
from jax.experimental import pallas as pl
from jax.experimental.pallas import tpu as pltpu


def _round_up(v, m):
    return ((v + m - 1) // m) * m


def _pad2(a, rows, cols):
    pr, pc = rows - a.shape[0], cols - a.shape[1]
    if pr or pc:
        a = jnp.pad(a, ((0, pr), (0, pc)))
    return a


def _weight_body(b_ref, a_ref, c_ref, w_ref, ab_ref):
    n = pl.program_id(0)

    @pl.when(n == 0)
    def _prep():
        ab_ref[...] = a_ref[...].astype(jnp.bfloat16)

    low = lax.dot_general(
        b_ref[...].astype(jnp.bfloat16), ab_ref[...],
        dimension_numbers=(((1,), (0,)), ((), ())),
        preferred_element_type=jnp.float32,
    )
    w_ref[...] = (c_ref[...] + low).astype(jnp.bfloat16)


def _gemm_body(x_ref, w_ref, bias_ref, o_ref, xs_ref):
    j = pl.program_id(1)

    @pl.when(j == 0)
    def _cast_x():
        xs_ref[...] = x_ref[...].astype(jnp.bfloat16)

    o_ref[...] = lax.dot_general(
        xs_ref[...], w_ref[...],
        dimension_numbers=(((1,), (1,)), ((), ())),
        preferred_element_type=jnp.float32,
    ) + bias_ref[...]


def kernel(x, weightA, weightB, weightC, bias):
    batch, in_f = x.shape
    out_f, rank = weightB.shape
    out_dtype = x.dtype

    tm = min(1024, _round_up(batch, 8))
    tn = min(512, _round_up(out_f, 128))
    tw = min(256, _round_up(out_f, 128))
    M = _round_up(batch, tm)
    N = _round_up(out_f, tn)
    K = _round_up(in_f, 128)
    R = _round_up(rank, 128)

    x_p = _pad2(x, M, K)
    a_p = _pad2(weightA, R, K)
    c_p = _pad2(weightC, N, K)
    b_p = _pad2(weightB, N, R)
    bias_p = _pad2(bias.reshape(1, out_f).astype(jnp.float32), 1, N)

    w_eff = pl.pallas_call(
        _weight_body,
        out_shape=jax.ShapeDtypeStruct((N, K), jnp.bfloat16),
        grid=(N // tw,),
        in_specs=[
            pl.BlockSpec((tw, R), lambda n: (n, 0)),
            pl.BlockSpec((R, K), lambda n: (0, 0)),
            pl.BlockSpec((tw, K), lambda n: (n, 0)),
        ],
        out_specs=pl.BlockSpec((tw, K), lambda n: (n, 0)),
        scratch_shapes=[
            pltpu.VMEM((R, K), jnp.bfloat16),
        ],
        compiler_params=pltpu.CompilerParams(
            dimension_semantics=("arbitrary",),
            vmem_limit_bytes=56 * 1024 * 1024,
        ),
    )(b_p, a_p, c_p)

    out = pl.pallas_call(
        _gemm_body,
        out_shape=jax.ShapeDtypeStruct((M, N), out_dtype),
        grid=(M // tm, N // tn),
        in_specs=[
            pl.BlockSpec((tm, K), lambda i, j: (i, 0)),
            pl.BlockSpec((tn, K), lambda i, j: (j, 0)),
            pl.BlockSpec((1, tn), lambda i, j: (0, j)),
        ],
        out_specs=pl.BlockSpec((tm, tn), lambda i, j: (i, j)),
        scratch_shapes=[
            pltpu.VMEM((tm, K), jnp.bfloat16),
        ],
        compiler_params=pltpu.CompilerParams(
            dimension_semantics=("parallel", "arbitrary"),
            vmem_limit_bytes=56 * 1024 * 1024,
        ),
    )(x_p, w_eff, bias_p)

    if M != batch or N != out_f:
        out = out[:batch, :out_f]
    return out


# speedup vs baseline: 1.0118x; 1.0080x over previous
import jax
import jax.numpy as jnp
from jax import lax
from jax.experimental import pallas as pl
from jax.experimental.pallas import tpu as pltpu


def _round_up(v, m):
    return ((v + m - 1) // m) * m


def _pad2(a, rows, cols):
    pr, pc = rows - a.shape[0], cols - a.shape[1]
    if pr or pc:
        a = jnp.pad(a, ((0, pr), (0, pc)))
    return a


def _weight_body(b_ref, a_ref, c_ref, w_ref, ab_ref):
    n = pl.program_id(0)

    @pl.when(n == 0)
    def _prep():
        ab_ref[...] = a_ref[...].astype(jnp.bfloat16)

    low = lax.dot_general(
        b_ref[...].astype(jnp.bfloat16), ab_ref[...],
        dimension_numbers=(((1,), (0,)), ((), ())),
        preferred_element_type=jnp.float32,
    )
    w_ref[...] = (c_ref[...] + low).astype(jnp.bfloat16)


def _gemm_body(x_ref, w_ref, bias_ref, o_ref, xs_ref):
    j = pl.program_id(1)

    @pl.when(j == 0)
    def _cast_x():
        xs_ref[...] = x_ref[...].astype(jnp.bfloat16)

    o_ref[...] = lax.dot_general(
        xs_ref[...], w_ref[...],
        dimension_numbers=(((1,), (1,)), ((), ())),
        preferred_element_type=jnp.float32,
    ) + bias_ref[...]


def kernel(x, weightA, weightB, weightC, bias):
    batch, in_f = x.shape
    out_f, rank = weightB.shape
    out_dtype = x.dtype

    tm = min(1024, _round_up(batch, 8))
    tn = min(512, _round_up(out_f, 128))
    tw = min(512, _round_up(out_f, 128))
    M = _round_up(batch, tm)
    N = _round_up(out_f, tn)
    K = _round_up(in_f, 128)
    R = _round_up(rank, 128)

    x_p = _pad2(x, M, K)
    a_p = _pad2(weightA, R, K)
    c_p = _pad2(weightC, N, K)
    b_p = _pad2(weightB, N, R)
    bias_p = _pad2(bias.reshape(1, out_f).astype(jnp.float32), 1, N)

    w_eff = pl.pallas_call(
        _weight_body,
        out_shape=jax.ShapeDtypeStruct((N, K), jnp.bfloat16),
        grid=(N // tw,),
        in_specs=[
            pl.BlockSpec((tw, R), lambda n: (n, 0)),
            pl.BlockSpec((R, K), lambda n: (0, 0)),
            pl.BlockSpec((tw, K), lambda n: (n, 0)),
        ],
        out_specs=pl.BlockSpec((tw, K), lambda n: (n, 0)),
        scratch_shapes=[
            pltpu.VMEM((R, K), jnp.bfloat16),
        ],
        compiler_params=pltpu.CompilerParams(
            dimension_semantics=("arbitrary",),
            vmem_limit_bytes=56 * 1024 * 1024,
        ),
    )(b_p, a_p, c_p)

    out = pl.pallas_call(
        _gemm_body,
        out_shape=jax.ShapeDtypeStruct((M, N), out_dtype),
        grid=(M // tm, N // tn),
        in_specs=[
            pl.BlockSpec((tm, K), lambda i, j: (i, 0)),
            pl.BlockSpec((tn, K), lambda i, j: (j, 0)),
            pl.BlockSpec((1, tn), lambda i, j: (0, j)),
        ],
        out_specs=pl.BlockSpec((tm, tn), lambda i, j: (i, j)),
        scratch_shapes=[
            pltpu.VMEM((tm, K), jnp.bfloat16),
        ],
        compiler_params=pltpu.CompilerParams(
            dimension_semantics=("parallel", "arbitrary"),
            vmem_limit_bytes=56 * 1024 * 1024,
        ),
    )(x_p, w_eff, bias_p)

    if M != batch or N != out_f:
        out = out[:batch, :out_f]
    return out


# speedup vs baseline: 1.0304x; 1.0185x over previous
import functools

import jax
import jax.numpy as jnp
from jax import lax
from jax.experimental import pallas as pl
from jax.experimental.pallas import tpu as pltpu


def _round_up(v, m):
    return ((v + m - 1) // m) * m


def _pad2(a, rows, cols):
    pr, pc = rows - a.shape[0], cols - a.shape[1]
    if pr or pc:
        a = jnp.pad(a, ((0, pr), (0, pc)))
    return a


def _weight_body(b_ref, a_ref, c_ref, w_ref, ab_ref):
    n = pl.program_id(0)

    @pl.when(n == 0)
    def _prep():
        ab_ref[...] = a_ref[...].astype(jnp.bfloat16)

    low = lax.dot_general(
        b_ref[...].astype(jnp.bfloat16), ab_ref[...],
        dimension_numbers=(((1,), (0,)), ((), ())),
        preferred_element_type=jnp.float32,
    )
    w_ref[...] = (c_ref[...] + low).astype(jnp.bfloat16)


def _gemm_body(x_ref, w_ref, bias_ref, o_ref, xs_ref, *, nj):
    j = pl.program_id(1)

    @pl.when(j == 0)
    def _cast_x():
        xs_ref[...] = x_ref[...].astype(jnp.bfloat16)

    i = pl.program_id(0)
    tn = w_ref.shape[0]
    jj = jnp.where(i % 2 == 0, j, nj - 1 - j)
    o_ref[...] = lax.dot_general(
        xs_ref[...], w_ref[...],
        dimension_numbers=(((1,), (1,)), ((), ())),
        preferred_element_type=jnp.float32,
    ) + bias_ref[:, pl.ds(jj * tn, tn)]


def kernel(x, weightA, weightB, weightC, bias):
    batch, in_f = x.shape
    out_f, rank = weightB.shape
    out_dtype = x.dtype

    tm = min(1024, _round_up(batch, 8))
    tn = min(512, _round_up(out_f, 128))
    tw = min(512, _round_up(out_f, 128))
    M = _round_up(batch, tm)
    N = _round_up(out_f, tn)
    K = _round_up(in_f, 128)
    R = _round_up(rank, 128)

    x_p = _pad2(x, M, K)
    a_p = _pad2(weightA, R, K)
    c_p = _pad2(weightC, N, K)
    b_p = _pad2(weightB, N, R)
    bias_p = _pad2(bias.reshape(1, out_f).astype(jnp.float32), 1, N)

    w_eff = pl.pallas_call(
        _weight_body,
        out_shape=jax.ShapeDtypeStruct((N, K), jnp.bfloat16),
        grid=(N // tw,),
        in_specs=[
            pl.BlockSpec((tw, R), lambda n: (n, 0)),
            pl.BlockSpec((R, K), lambda n: (0, 0)),
            pl.BlockSpec((tw, K), lambda n: (n, 0)),
        ],
        out_specs=pl.BlockSpec((tw, K), lambda n: (n, 0)),
        scratch_shapes=[
            pltpu.VMEM((R, K), jnp.bfloat16),
        ],
        compiler_params=pltpu.CompilerParams(
            dimension_semantics=("arbitrary",),
            vmem_limit_bytes=56 * 1024 * 1024,
        ),
    )(b_p, a_p, c_p)

    nj = N // tn

    def _serp(i, j):
        return jnp.where(i % 2 == 0, j, nj - 1 - j)

    out = pl.pallas_call(
        functools.partial(_gemm_body, nj=nj),
        out_shape=jax.ShapeDtypeStruct((M, N), out_dtype),
        grid=(M // tm, nj),
        in_specs=[
            pl.BlockSpec((tm, K), lambda i, j: (i, 0)),
            pl.BlockSpec((tn, K), lambda i, j: (_serp(i, j), 0)),
            pl.BlockSpec((1, N), lambda i, j: (0, 0)),
        ],
        out_specs=pl.BlockSpec((tm, tn), lambda i, j: (i, _serp(i, j))),
        scratch_shapes=[
            pltpu.VMEM((tm, K), jnp.bfloat16),
        ],
        compiler_params=pltpu.CompilerParams(
            dimension_semantics=("parallel", "arbitrary"),
            vmem_limit_bytes=56 * 1024 * 1024,
        ),
    )(x_p, w_eff, bias_p)

    if M != batch or N != out_f:
        out = out[:batch, :out_f]
    return out


# speedup vs baseline: 1.0426x; 1.0118x over previous
import functools

import jax
import jax.numpy as jnp
from jax import lax
from jax.experimental import pallas as pl
from jax.experimental.pallas import tpu as pltpu


def _round_up(v, m):
    return ((v + m - 1) // m) * m


def _pad2(a, rows, cols):
    pr, pc = rows - a.shape[0], cols - a.shape[1]
    if pr or pc:
        a = jnp.pad(a, ((0, pr), (0, pc)))
    return a


def _weight_body(b_ref, a_ref, c_ref, w_ref, ab_ref):
    n = pl.program_id(0)
    tw = c_ref.shape[0]

    @pl.when(n == 0)
    def _prep():
        ab_ref[...] = a_ref[...].astype(jnp.bfloat16)

    low = lax.dot_general(
        b_ref[pl.ds(n * tw, tw), :].astype(jnp.bfloat16), ab_ref[...],
        dimension_numbers=(((1,), (0,)), ((), ())),
        preferred_element_type=jnp.float32,
    )
    w_ref[...] = (c_ref[...] + low).astype(jnp.bfloat16)


def _gemm_body(x_ref, w_ref, bias_ref, o_ref, xs_ref, *, nj):
    j = pl.program_id(1)

    @pl.when(j == 0)
    def _cast_x():
        xs_ref[...] = x_ref[...].astype(jnp.bfloat16)

    i = pl.program_id(0)
    tn = w_ref.shape[0]
    jj = jnp.where(i % 2 == 0, j, nj - 1 - j)
    o_ref[...] = lax.dot_general(
        xs_ref[...], w_ref[...],
        dimension_numbers=(((1,), (1,)), ((), ())),
        preferred_element_type=jnp.float32,
    ) + bias_ref[:, pl.ds(jj * tn, tn)]


def kernel(x, weightA, weightB, weightC, bias):
    batch, in_f = x.shape
    out_f, rank = weightB.shape
    out_dtype = x.dtype

    tm = min(1024, _round_up(batch, 8))
    tn = min(512, _round_up(out_f, 128))
    tw = min(512, _round_up(out_f, 128))
    M = _round_up(batch, tm)
    N = _round_up(out_f, tn)
    K = _round_up(in_f, 128)
    R = _round_up(rank, 128)

    x_p = _pad2(x, M, K)
    a_p = _pad2(weightA, R, K)
    c_p = _pad2(weightC, N, K)
    b_p = _pad2(weightB, N, R)
    bias_p = _pad2(bias.reshape(1, out_f).astype(jnp.float32), 1, N)

    w_eff = pl.pallas_call(
        _weight_body,
        out_shape=jax.ShapeDtypeStruct((N, K), jnp.bfloat16),
        grid=(N // tw,),
        in_specs=[
            pl.BlockSpec((N, R), lambda n: (0, 0)),
            pl.BlockSpec((R, K), lambda n: (0, 0)),
            pl.BlockSpec((tw, K), lambda n: (n, 0)),
        ],
        out_specs=pl.BlockSpec((tw, K), lambda n: (n, 0)),
        scratch_shapes=[
            pltpu.VMEM((R, K), jnp.bfloat16),
        ],
        compiler_params=pltpu.CompilerParams(
            dimension_semantics=("arbitrary",),
            vmem_limit_bytes=56 * 1024 * 1024,
        ),
    )(b_p, a_p, c_p)

    nj = N // tn

    def _serp(i, j):
        return jnp.where(i % 2 == 0, j, nj - 1 - j)

    out = pl.pallas_call(
        functools.partial(_gemm_body, nj=nj),
        out_shape=jax.ShapeDtypeStruct((M, N), out_dtype),
        grid=(M // tm, nj),
        in_specs=[
            pl.BlockSpec((tm, K), lambda i, j: (i, 0)),
            pl.BlockSpec((tn, K), lambda i, j: (_serp(i, j), 0)),
            pl.BlockSpec((1, N), lambda i, j: (0, 0)),
        ],
        out_specs=pl.BlockSpec((tm, tn), lambda i, j: (i, _serp(i, j))),
        scratch_shapes=[
            pltpu.VMEM((tm, K), jnp.bfloat16),
        ],
        compiler_params=pltpu.CompilerParams(
            dimension_semantics=("parallel", "arbitrary"),
            vmem_limit_bytes=56 * 1024 * 1024,
        ),
    )(x_p, w_eff, bias_p)

    if M != batch or N != out_f:
        out = out[:batch, :out_f]
    return out


# speedup vs baseline: 1.0865x; 1.0421x over previous
import functools

import jax
import jax.numpy as jnp
from jax import lax
from jax.experimental import pallas as pl
from jax.experimental.pallas import tpu as pltpu


def _round_up(v, m):
    return ((v + m - 1) // m) * m


def _pad2(a, rows, cols):
    pr, pc = rows - a.shape[0], cols - a.shape[1]
    if pr or pc:
        a = jnp.pad(a, ((0, pr), (0, pc)))
    return a


def _weight_body(b_ref, a_ref, c_ref, w_ref, ab_ref):
    n = pl.program_id(0)
    tw = c_ref.shape[0]

    @pl.when(n == 0)
    def _prep():
        ab_ref[...] = a_ref[...].astype(jnp.bfloat16)

    low = lax.dot_general(
        b_ref[pl.ds(n * tw, tw), :].astype(jnp.bfloat16), ab_ref[...],
        dimension_numbers=(((1,), (0,)), ((), ())),
        preferred_element_type=jnp.float32,
    )
    w_ref[...] = (c_ref[...] + low).astype(jnp.bfloat16)


def _gemm_body(x_ref, w_ref, bias_ref, o_ref, xs_ref, *, nj):
    j = pl.program_id(1)

    @pl.when(j == 0)
    def _cast_x():
        xs_ref[...] = x_ref[...].astype(jnp.bfloat16)

    i = pl.program_id(0)
    tn = w_ref.shape[0]
    jj = jnp.where(i % 2 == 0, j, nj - 1 - j)
    o_ref[...] = lax.dot_general(
        xs_ref[...], w_ref[...],
        dimension_numbers=(((1,), (1,)), ((), ())),
        preferred_element_type=jnp.float32,
    ) + bias_ref[:, pl.ds(jj * tn, tn)]


def kernel(x, weightA, weightB, weightC, bias):
    batch, in_f = x.shape
    out_f, rank = weightB.shape
    out_dtype = x.dtype

    tm = min(512, _round_up(batch, 8))
    tn = min(1024, _round_up(out_f, 128))
    tw = min(512, _round_up(out_f, 128))
    M = _round_up(batch, tm)
    N = _round_up(out_f, tn)
    K = _round_up(in_f, 128)
    R = _round_up(rank, 128)

    x_p = _pad2(x, M, K)
    a_p = _pad2(weightA, R, K)
    c_p = _pad2(weightC, N, K)
    b_p = _pad2(weightB, N, R)
    bias_p = _pad2(bias.reshape(1, out_f).astype(jnp.float32), 1, N)

    w_eff = pl.pallas_call(
        _weight_body,
        out_shape=jax.ShapeDtypeStruct((N, K), jnp.bfloat16),
        grid=(N // tw,),
        in_specs=[
            pl.BlockSpec((N, R), lambda n: (0, 0)),
            pl.BlockSpec((R, K), lambda n: (0, 0)),
            pl.BlockSpec((tw, K), lambda n: (n, 0)),
        ],
        out_specs=pl.BlockSpec((tw, K), lambda n: (n, 0)),
        scratch_shapes=[
            pltpu.VMEM((R, K), jnp.bfloat16),
        ],
        compiler_params=pltpu.CompilerParams(
            dimension_semantics=("arbitrary",),
            vmem_limit_bytes=56 * 1024 * 1024,
        ),
    )(b_p, a_p, c_p)

    nj = N // tn

    def _serp(i, j):
        return jnp.where(i % 2 == 0, j, nj - 1 - j)

    out = pl.pallas_call(
        functools.partial(_gemm_body, nj=nj),
        out_shape=jax.ShapeDtypeStruct((M, N), out_dtype),
        grid=(M // tm, nj),
        in_specs=[
            pl.BlockSpec((tm, K), lambda i, j: (i, 0)),
            pl.BlockSpec((tn, K), lambda i, j: (_serp(i, j), 0)),
            pl.BlockSpec((1, N), lambda i, j: (0, 0)),
        ],
        out_specs=pl.BlockSpec((tm, tn), lambda i, j: (i, _serp(i, j))),
        scratch_shapes=[
            pltpu.VMEM((tm, K), jnp.bfloat16),
        ],
        compiler_params=pltpu.CompilerParams(
            dimension_semantics=("parallel", "arbitrary"),
            vmem_limit_bytes=56 * 1024 * 1024,
        ),
    )(x_p, w_eff, bias_p)

    if M != batch or N != out_f:
        out = out[:batch, :out_f]
    return out


# speedup vs baseline: 1.1043x; 1.0164x over previous
import functools

import jax
import jax.numpy as jnp
from jax import lax
from jax.experimental import pallas as pl
from jax.experimental.pallas import tpu as pltpu


def _round_up(v, m):
    return ((v + m - 1) // m) * m


def _pad2(a, rows, cols):
    pr, pc = rows - a.shape[0], cols - a.shape[1]
    if pr or pc:
        a = jnp.pad(a, ((0, pr), (0, pc)))
    return a


def _weight_body(b_ref, a_ref, c_ref, w_ref, ab_ref):
    n = pl.program_id(0)
    tw = c_ref.shape[0]

    @pl.when(n == 0)
    def _prep():
        ab_ref[...] = a_ref[...].astype(jnp.bfloat16)

    low = lax.dot_general(
        b_ref[pl.ds(n * tw, tw), :].astype(jnp.bfloat16), ab_ref[...],
        dimension_numbers=(((1,), (0,)), ((), ())),
        preferred_element_type=jnp.float32,
    )
    w_ref[...] = (c_ref[...] + low).astype(jnp.bfloat16)


def _gemm_body(x_ref, w_ref, bias_ref, o_ref, xs_ref, *, nj):
    j = pl.program_id(1)

    @pl.when(j == 0)
    def _cast_x():
        xs_ref[...] = x_ref[...].astype(jnp.bfloat16)

    i = pl.program_id(0)
    tn = w_ref.shape[0]
    jj = jnp.where(i % 2 == 0, j, nj - 1 - j)
    o_ref[...] = lax.dot_general(
        xs_ref[...], w_ref[...],
        dimension_numbers=(((1,), (1,)), ((), ())),
        preferred_element_type=jnp.float32,
    ) + bias_ref[:, pl.ds(jj * tn, tn)]


def kernel(x, weightA, weightB, weightC, bias):
    batch, in_f = x.shape
    out_f, rank = weightB.shape
    out_dtype = x.dtype

    tm = min(512, _round_up(batch, 8))
    tn = min(2048, _round_up(out_f, 128))
    tw = min(512, _round_up(out_f, 128))
    M = _round_up(batch, tm)
    N = _round_up(out_f, tn)
    K = _round_up(in_f, 128)
    R = _round_up(rank, 128)

    x_p = _pad2(x, M, K)
    a_p = _pad2(weightA, R, K)
    c_p = _pad2(weightC, N, K)
    b_p = _pad2(weightB, N, R)
    bias_p = _pad2(bias.reshape(1, out_f).astype(jnp.float32), 1, N)

    w_eff = pl.pallas_call(
        _weight_body,
        out_shape=jax.ShapeDtypeStruct((N, K), jnp.bfloat16),
        grid=(N // tw,),
        in_specs=[
            pl.BlockSpec((N, R), lambda n: (0, 0)),
            pl.BlockSpec((R, K), lambda n: (0, 0)),
            pl.BlockSpec((tw, K), lambda n: (n, 0)),
        ],
        out_specs=pl.BlockSpec((tw, K), lambda n: (n, 0)),
        scratch_shapes=[
            pltpu.VMEM((R, K), jnp.bfloat16),
        ],
        compiler_params=pltpu.CompilerParams(
            dimension_semantics=("arbitrary",),
            vmem_limit_bytes=56 * 1024 * 1024,
        ),
    )(b_p, a_p, c_p)

    nj = N // tn

    def _serp(i, j):
        return jnp.where(i % 2 == 0, j, nj - 1 - j)

    out = pl.pallas_call(
        functools.partial(_gemm_body, nj=nj),
        out_shape=jax.ShapeDtypeStruct((M, N), out_dtype),
        grid=(M // tm, nj),
        in_specs=[
            pl.BlockSpec((tm, K), lambda i, j: (i, 0)),
            pl.BlockSpec((tn, K), lambda i, j: (_serp(i, j), 0)),
            pl.BlockSpec((1, N), lambda i, j: (0, 0)),
        ],
        out_specs=pl.BlockSpec((tm, tn), lambda i, j: (i, _serp(i, j))),
        scratch_shapes=[
            pltpu.VMEM((tm, K), jnp.bfloat16),
        ],
        compiler_params=pltpu.CompilerParams(
            dimension_semantics=("parallel", "arbitrary"),
            vmem_limit_bytes=64 * 1024 * 1024,
        ),
    )(x_p, w_eff, bias_p)

    if M != batch or N != out_f:
        out = out[:batch, :out_f]
    return out
